# baseline (speedup 1.0000x reference)
import functools

import jax
import jax.numpy as jnp
from jax import lax
from jax.experimental import pallas as pl
from jax.experimental.pallas import tpu as pltpu
from jax.experimental.pallas import tpu_sc as plsc

N = 16384
DIM = 64
OUT_D = DIM + 2

_info = plsc.get_sparse_core_info()
NC = _info.num_cores
NS = _info.num_subcores
L = _info.num_lanes
NW = NC * NS
B_PER_W = N // NW
CHUNK = 128
N_CHUNKS = B_PER_W // CHUNK


def _emb_kernel(vocab_hbm, sel_hbm, st_hbm, table_hbm,
                out_hbm, cola_hbm, colb_hbm,
                idx_v, sel_v, st_v, out_v, cola_v, colb_v, sem):
    wid = lax.axis_index("s") * NC + lax.axis_index("c")
    base = wid * B_PER_W

    pltpu.sync_copy(vocab_hbm.at[pl.ds(base, B_PER_W)], idx_v)
    pltpu.sync_copy(sel_hbm.at[pl.ds(base, B_PER_W)], sel_v)
    pltpu.sync_copy(st_hbm, st_v)

    copies = []
    for c in range(N_CHUNKS):
        copies.append(
            pltpu.async_copy(
                table_hbm.at[idx_v.at[pl.ds(c * CHUNK, CHUNK)]],
                out_v.at[pl.ds(c * CHUNK, CHUNK), :],
                sem,
            )
        )

    st00 = st_v[pl.ds(0 * L, L)]
    st01 = st_v[pl.ds(1 * L, L)]
    st10 = st_v[pl.ds(2 * L, L)]
    st11 = st_v[pl.ds(3 * L, L)]

    def body(i, carry):
        s = sel_v[pl.ds(i * L, L)]
        is0 = s == 0
        cola_v[pl.ds(i * L, L)] = jnp.where(is0, st00, st10)
        colb_v[pl.ds(i * L, L)] = jnp.where(is0, st01, st11)
        return carry

    lax.fori_loop(0, B_PER_W // L, body, 0)

    pltpu.sync_copy(cola_v, cola_hbm.at[pl.ds(base, B_PER_W)])
    pltpu.sync_copy(colb_v, colb_hbm.at[pl.ds(base, B_PER_W)])

    for cp in copies:
        cp.wait()

    pltpu.sync_copy(out_v, out_hbm.at[pl.ds(base, B_PER_W)])


@jax.jit
def _emb(vocab_ids, selector_ids, table, selector_table):
    mesh = plsc.VectorSubcoreMesh(core_axis_name="c", subcore_axis_name="s")
    st64 = jnp.repeat(selector_table.reshape(-1), L)
    f = functools.partial(
        pl.kernel,
        mesh=mesh,
        out_type=(
            jax.ShapeDtypeStruct((N, DIM), jnp.float32),
            jax.ShapeDtypeStruct((N,), jnp.float32),
            jax.ShapeDtypeStruct((N,), jnp.float32),
        ),
        scratch_types=[
            pltpu.VMEM((B_PER_W,), jnp.int32),
            pltpu.VMEM((B_PER_W,), jnp.int32),
            pltpu.VMEM((4 * L,), jnp.float32),
            pltpu.VMEM((B_PER_W, DIM), jnp.float32),
            pltpu.VMEM((B_PER_W,), jnp.float32),
            pltpu.VMEM((B_PER_W,), jnp.float32),
            pltpu.SemaphoreType.DMA,
        ],
        compiler_params=pltpu.CompilerParams(use_tc_tiling_on_sc=False),
    )(_emb_kernel)
    rows, cola, colb = f(vocab_ids, selector_ids, st64, table)
    return jnp.concatenate(
        (rows, cola[:, None], colb[:, None]), axis=1)


def kernel(vocab_ids, selector_ids, table, selector_table):
    return _emb(vocab_ids.astype(jnp.int32), selector_ids.astype(jnp.int32),
                table, selector_table)
